# baseline (speedup 1.0000x reference)
import jax
import jax.numpy as jnp
from jax import lax
from jax.experimental import pallas as pl
from jax.experimental.pallas import tpu as pltpu
from jax.experimental.pallas import tpu_sc as plsc

N = 10000
NPAD = 10240
E = 320000
D = 128
NP2 = 320000
EPS = 1e-5

NC = 2
NS = 16
NW = NC * NS

CH = 80
EPT = E // NW
NCH = EPT // CH
PPT = NP2 // NW
RPT = NPAD // NS

_f32 = jnp.float32
_i32 = jnp.int32


def _mesh():
    return plsc.VectorSubcoreMesh(core_axis_name="c", subcore_axis_name="s",
                                  num_cores=NC, num_subcores=NS)


def _deg_body(w, dst, zeros, out, wv0, wv1, dstv0, dstv1, acc, semi, semw):
    cc = lax.axis_index("c")
    sid = lax.axis_index("s")
    wid = cc * NS + sid
    wv = (wv0, wv1)
    dstv = (dstv0, dstv1)
    eoff = wid * EPT

    @pl.when(sid == 0)
    def _():
        pltpu.sync_copy(zeros, acc)
    plsc.subcore_barrier()

    def stage(c, b):
        pltpu.async_copy(w.at[pl.ds(eoff + c * CH, CH)], wv[b], semi.at[b])
        pltpu.async_copy(dst.at[pl.ds(eoff + c * CH, CH)], dstv[b],
                         semi.at[b])

    def wait_stage(b):
        pltpu.make_async_copy(w.at[pl.ds(eoff, CH)], wv[b], semi.at[b]).wait()
        pltpu.make_async_copy(dst.at[pl.ds(eoff, CH)], dstv[b],
                              semi.at[b]).wait()

    def scatter(b):
        pltpu.async_copy(wv[b], acc.at[dstv[b]], semw.at[b], add=True)

    def wait_scatter(b):
        pltpu.make_async_copy(wv[b], acc.at[dstv[b]], semw.at[b]).wait()

    stage(0, 0)
    wait_stage(0)
    scatter(0)
    stage(1, 1)

    def pair(t, carry):
        c1 = 2 * t + 1
        wait_stage(1)
        wait_scatter(0)
        scatter(1)
        stage(c1 + 1, 0)

        c2 = 2 * t + 2
        wait_stage(0)
        wait_scatter(1)
        scatter(0)

        @pl.when(t < (NCH - 3) // 2)
        def _():
            stage(c2 + 1, 1)
        return carry
    lax.fori_loop(0, (NCH - 1) // 2, pair, 0)
    wait_scatter(0)

    plsc.subcore_barrier()

    @pl.when(sid == 0)
    def _():
        pltpu.sync_copy(acc, out.at[cc])


def _sc_deg(w, dst, zeros):
    return pl.kernel(
        _deg_body,
        out_type=jax.ShapeDtypeStruct((NC, N), _f32),
        mesh=_mesh(),
        scratch_types=[
            pltpu.VMEM((CH,), _f32),
            pltpu.VMEM((CH,), _f32),
            pltpu.VMEM((CH,), _i32),
            pltpu.VMEM((CH,), _i32),
            pltpu.VMEM_SHARED((N,), _f32),
            pltpu.SemaphoreType.DMA((2,)),
            pltpu.SemaphoreType.DMA((2,)),
        ],
    )(w, dst, zeros)


def _mp_body(y, src, dst, w, out, srca, wa, rows0, rows1, idxs0, idxs1, acc,
             semg, semi, semw):
    cc = lax.axis_index("c")
    sid = lax.axis_index("s")
    wid = cc * NS + sid
    rows = (rows0, rows1)
    idxs = (idxs0, idxs1)

    def zr(i, carry):
        for j in range(8):
            rows0[i, pl.ds(j * 16, 16)] = jnp.zeros((16,), _f32)
        return carry
    lax.fori_loop(0, CH, zr, 0)

    base = sid * RPT
    for k in range(RPT // CH):
        pltpu.sync_copy(rows0, acc.at[pl.ds(base + k * CH, CH)])
    plsc.subcore_barrier()

    eoff = wid * EPT
    pltpu.sync_copy(src.at[pl.ds(eoff, EPT)], srca)
    pltpu.sync_copy(w.at[pl.ds(eoff, EPT)], wa)

    def issue_next(c, b):
        pltpu.async_copy(dst.at[pl.ds(eoff + c * CH, CH)], idxs[b],
                         semi.at[b])
        pltpu.async_copy(y.at[srca.at[pl.ds(c * CH, CH)]], rows[b],
                         semg.at[b])

    def wait_in(b):
        pltpu.make_async_copy(dst.at[pl.ds(eoff, CH)], idxs[b],
                              semi.at[b]).wait()
        pltpu.make_async_copy(y.at[srca.at[pl.ds(0, CH)]], rows[b],
                              semg.at[b]).wait()

    def scatter(b):
        pltpu.async_copy(rows[b], acc.at[idxs[b]], semw.at[b], add=True)

    def wait_scatter(b):
        pltpu.make_async_copy(rows[b], acc.at[idxs[b]], semw.at[b]).wait()

    def scale(c, b):
        rb = rows[b]

        def group(g, c2):
            wvec = wa[pl.ds(c * CH + g * 16, 16)]
            for lane in range(16):
                ws = wvec[lane]
                e = g * 16 + lane
                for j in range(8):
                    s = pl.ds(j * 16, 16)
                    rb[e, s] = rb[e, s] * ws
            return c2
        lax.fori_loop(0, CH // 16, group, 0)

    issue_next(0, 0)
    wait_in(0)
    scale(0, 0)
    scatter(0)
    issue_next(1, 1)

    def pair(t, carry):
        c1 = 2 * t + 1
        wait_in(1)
        scale(c1, 1)
        wait_scatter(0)
        scatter(1)
        issue_next(c1 + 1, 0)

        c2 = 2 * t + 2
        wait_in(0)
        scale(c2, 0)
        wait_scatter(1)
        scatter(0)

        @pl.when(t < (NCH - 3) // 2)
        def _():
            issue_next(c2 + 1, 1)
        return carry
    lax.fori_loop(0, (NCH - 1) // 2, pair, 0)
    wait_scatter(0)

    plsc.subcore_barrier()
    pltpu.sync_copy(acc.at[pl.ds(base, RPT)], out.at[cc, pl.ds(base, RPT)])


def _sc_mp(y, src, dst, w):
    return pl.kernel(
        _mp_body,
        out_type=jax.ShapeDtypeStruct((NC, NPAD, D), _f32),
        mesh=_mesh(),
        scratch_types=[
            pltpu.VMEM((EPT,), _i32),
            pltpu.VMEM((EPT,), _f32),
            pltpu.VMEM((CH, D), _f32),
            pltpu.VMEM((CH, D), _f32),
            pltpu.VMEM((CH,), _i32),
            pltpu.VMEM((CH,), _i32),
            pltpu.VMEM_SHARED((NPAD, D), _f32),
            pltpu.SemaphoreType.DMA((2,)),
            pltpu.SemaphoreType.DMA((2,)),
            pltpu.SemaphoreType.DMA((2,)),
        ],
    )(y, src, dst, w)


_DCH = PPT // CH


def _dec_body(p, q, i0, i1, out, i0a, i1a, ra0, ra1, semg, sema, semw):
    cc = lax.axis_index("c")
    sid = lax.axis_index("s")
    wid = cc * NS + sid
    poff = wid * PPT
    ra = (ra0, ra1)

    pltpu.sync_copy(i0.at[pl.ds(poff, PPT)], i0a)
    pltpu.sync_copy(i1.at[pl.ds(poff, PPT)], i1a)

    def g1(c, b):
        pltpu.async_copy(p.at[i0a.at[pl.ds(c * CH, CH)]], ra[b], semg.at[b])

    def wait_g1(b):
        pltpu.make_async_copy(p.at[i0a.at[pl.ds(0, CH)]], ra[b],
                              semg.at[b]).wait()

    def gadd(c, b):
        pltpu.async_copy(q.at[i1a.at[pl.ds(c * CH, CH)]], ra[b], sema.at[b],
                         add=True)

    def wait_gadd(b):
        pltpu.make_async_copy(q.at[i1a.at[pl.ds(0, CH)]], ra[b],
                              sema.at[b]).wait()

    def write(c, b):
        pltpu.async_copy(ra[b], out.at[pl.ds(poff + c * CH, CH)], semw.at[b])

    def wait_write(b):
        pltpu.make_async_copy(ra[b], out.at[pl.ds(poff, CH)],
                              semw.at[b]).wait()

    g1(0, 0)
    wait_g1(0)
    gadd(0, 0)
    g1(1, 1)
    wait_gadd(0)
    write(0, 0)

    def pair(t, carry):
        c1 = 2 * t + 1
        wait_g1(1)
        gadd(c1, 1)
        wait_write(0)
        g1(c1 + 1, 0)
        wait_gadd(1)
        write(c1, 1)

        c2 = 2 * t + 2
        wait_g1(0)
        gadd(c2, 0)
        wait_write(1)

        @pl.when(t < (_DCH - 3) // 2)
        def _():
            g1(c2 + 1, 1)
        wait_gadd(0)
        write(c2, 0)
        return carry
    lax.fori_loop(0, (_DCH - 1) // 2, pair, 0)
    wait_write(0)


def _sc_dec(p, q, i0, i1):
    return pl.kernel(
        _dec_body,
        out_type=jax.ShapeDtypeStruct((NP2, D), _f32),
        mesh=_mesh(),
        scratch_types=[
            pltpu.VMEM((PPT,), _i32),
            pltpu.VMEM((PPT,), _i32),
            pltpu.VMEM((CH, D), _f32),
            pltpu.VMEM((CH, D), _f32),
            pltpu.SemaphoreType.DMA((2,)),
            pltpu.SemaphoreType.DMA((2,)),
            pltpu.SemaphoreType.DMA((2,)),
        ],
    )(p, q, i0, i1)


def _tc_xw_body(x, w1, xw_o):
    xw_o[...] = jnp.dot(x[...], w1[...], preferred_element_type=_f32)


def _tc_xw(x, w1):
    return pl.pallas_call(
        _tc_xw_body,
        out_shape=jax.ShapeDtypeStruct((NPAD, D), _f32),
    )(x, w1)


def _tc_first_body(degp, xw, dis_o, y_o):
    dis = lax.rsqrt(1.0 + degp[0] + degp[1])
    dis_o[...] = dis
    y_o[...] = xw[...] * dis


def _tc_first(degp, xw):
    return pl.pallas_call(
        _tc_first_body,
        out_shape=(jax.ShapeDtypeStruct((NPAD, 1), _f32),
                   jax.ShapeDtypeStruct((NPAD, D), _f32)),
    )(degp, xw)


def _tc_mid_body(m, y, dis, b, g, beta, w, y_o):
    h = (m[0] + m[1] + y[...]) * dis[...] + b[...]
    hv = h[:N]
    mu = jnp.mean(hv, axis=0, keepdims=True)
    var = jnp.mean((hv - mu) ** 2, axis=0, keepdims=True)
    hn = (h - mu) * lax.rsqrt(var + EPS) * g[...] + beta[...]
    hn = jnp.maximum(hn, 0.0)
    y_o[...] = jnp.dot(hn, w[...], preferred_element_type=_f32) * dis[...]


def _tc_mid(m, y, dis, b, g, beta, w):
    return pl.pallas_call(
        _tc_mid_body,
        out_shape=jax.ShapeDtypeStruct((NPAD, D), _f32),
    )(m, y, dis, b, g, beta, w)


def _tc_enc_body(m, y, dis, b, dw1a, dw1b, db1, p_o, q_o):
    enc = (m[0] + m[1] + y[...]) * dis[...] + b[...]
    p_o[...] = jnp.dot(enc, dw1a[...], preferred_element_type=_f32) + db1[...]
    q_o[...] = jnp.dot(enc, dw1b[...], preferred_element_type=_f32)


def _tc_enc(m, y, dis, b, dw1a, dw1b, db1):
    return pl.pallas_call(
        _tc_enc_body,
        out_shape=(jax.ShapeDtypeStruct((NPAD, D), _f32),
                   jax.ShapeDtypeStruct((NPAD, D), _f32)),
    )(m, y, dis, b, dw1a, dw1b, db1)


_DEC_R = 1280


def _tc_dec_body(h1, dw2, db2, dw3, db3, o):
    h = jnp.maximum(h1[...], 0.0).astype(jnp.bfloat16)
    t = jnp.dot(h, dw2[...], preferred_element_type=_f32) + db2[...]
    t = jnp.maximum(t, 0.0).astype(jnp.bfloat16)
    o[...] = jnp.dot(t, dw3[...], preferred_element_type=_f32) + db3[...]


def _tc_dec(h1, dw2, db2, dw3, db3):
    grid = NP2 // _DEC_R
    return pl.pallas_call(
        _tc_dec_body,
        grid=(grid,),
        in_specs=[
            pl.BlockSpec((_DEC_R, D), lambda i: (i, 0)),
            pl.BlockSpec((D, D), lambda i: (0, 0)),
            pl.BlockSpec((1, D), lambda i: (0, 0)),
            pl.BlockSpec((D, 1), lambda i: (0, 0)),
            pl.BlockSpec((1, 1), lambda i: (0, 0)),
        ],
        out_specs=pl.BlockSpec((_DEC_R, 1), lambda i: (i, 0)),
        out_shape=jax.ShapeDtypeStruct((NP2, 1), _f32),
    )(h1, dw2, db2, dw3, db3)


def kernel(x, edge_weight, W1, b1, W2, b2, W3, b3, g1, beta1, g2, beta2,
           dW1, db1, dW2, db2, dW3, db3, edge_index,
           edge_label_index_only, neg_edge_index):
    src = edge_index[0].astype(_i32)
    dst = edge_index[1].astype(_i32)
    i0 = jnp.concatenate(
        [edge_label_index_only[0], neg_edge_index[0]]).astype(_i32)
    i1 = jnp.concatenate(
        [edge_label_index_only[1], neg_edge_index[1]]).astype(_i32)

    xp = jnp.pad(x, ((0, NPAD - N), (0, 0)))

    xw1 = _tc_xw(xp, W1)
    degp = _sc_deg(edge_weight, dst, jnp.zeros((N,), _f32))
    degp = jnp.pad(degp, ((0, 0), (0, NPAD - N))).reshape(NC, NPAD, 1)
    dis, y1 = _tc_first(degp, xw1)

    m1 = _sc_mp(y1, src, dst, edge_weight)
    y2 = _tc_mid(m1, y1, dis, b1.reshape(1, D), g1.reshape(1, D),
                 beta1.reshape(1, D), W2)
    m2 = _sc_mp(y2, src, dst, edge_weight)
    y3 = _tc_mid(m2, y2, dis, b2.reshape(1, D), g2.reshape(1, D),
                 beta2.reshape(1, D), W3)
    m3 = _sc_mp(y3, src, dst, edge_weight)

    p, q = _tc_enc(m3, y3, dis, b3.reshape(1, D), dW1[:D], dW1[D:],
                   db1.reshape(1, D))
    h1 = _sc_dec(p, q, i0, i1)
    out = _tc_dec(h1, dW2.astype(jnp.bfloat16), db2.reshape(1, D),
                  dW3.astype(jnp.bfloat16), db3.reshape(1, 1))
    return out[:, 0]


# speedup vs baseline: 1.0313x; 1.0313x over previous
import jax
import jax.numpy as jnp
from jax import lax
from jax.experimental import pallas as pl
from jax.experimental.pallas import tpu as pltpu
from jax.experimental.pallas import tpu_sc as plsc

N = 10000
NPAD = 10240
E = 320000
D = 128
NP2 = 320000
EPS = 1e-5

NC = 2
NS = 16
NW = NC * NS

CH = 80
EPT = E // NW
NCH = EPT // CH
PPT = NP2 // NW
RPT = NPAD // NS

_f32 = jnp.float32
_i32 = jnp.int32


def _mesh():
    return plsc.VectorSubcoreMesh(core_axis_name="c", subcore_axis_name="s",
                                  num_cores=NC, num_subcores=NS)


def _deg_body(w, dst, zeros, out, wv0, wv1, dstv0, dstv1, acc, semi, semw):
    cc = lax.axis_index("c")
    sid = lax.axis_index("s")
    wid = cc * NS + sid
    wv = (wv0, wv1)
    dstv = (dstv0, dstv1)
    eoff = wid * EPT

    @pl.when(sid == 0)
    def _():
        pltpu.sync_copy(zeros, acc)
    plsc.subcore_barrier()

    def stage(c, b):
        pltpu.async_copy(w.at[pl.ds(eoff + c * CH, CH)], wv[b], semi.at[b])
        pltpu.async_copy(dst.at[pl.ds(eoff + c * CH, CH)], dstv[b],
                         semi.at[b])

    def wait_stage(b):
        pltpu.make_async_copy(w.at[pl.ds(eoff, CH)], wv[b], semi.at[b]).wait()
        pltpu.make_async_copy(dst.at[pl.ds(eoff, CH)], dstv[b],
                              semi.at[b]).wait()

    def scatter(b):
        pltpu.async_copy(wv[b], acc.at[dstv[b]], semw.at[b], add=True)

    def wait_scatter(b):
        pltpu.make_async_copy(wv[b], acc.at[dstv[b]], semw.at[b]).wait()

    stage(0, 0)
    wait_stage(0)
    scatter(0)
    stage(1, 1)

    def pair(t, carry):
        c1 = 2 * t + 1
        wait_stage(1)
        wait_scatter(0)
        scatter(1)
        stage(c1 + 1, 0)

        c2 = 2 * t + 2
        wait_stage(0)
        wait_scatter(1)
        scatter(0)

        @pl.when(t < (NCH - 3) // 2)
        def _():
            stage(c2 + 1, 1)
        return carry
    lax.fori_loop(0, (NCH - 1) // 2, pair, 0)
    wait_scatter(0)

    plsc.subcore_barrier()

    @pl.when(sid == 0)
    def _():
        pltpu.sync_copy(acc, out.at[cc])


def _sc_deg(w, dst, zeros):
    return pl.kernel(
        _deg_body,
        out_type=jax.ShapeDtypeStruct((NC, N), _f32),
        mesh=_mesh(),
        scratch_types=[
            pltpu.VMEM((CH,), _f32),
            pltpu.VMEM((CH,), _f32),
            pltpu.VMEM((CH,), _i32),
            pltpu.VMEM((CH,), _i32),
            pltpu.VMEM_SHARED((N,), _f32),
            pltpu.SemaphoreType.DMA((2,)),
            pltpu.SemaphoreType.DMA((2,)),
        ],
    )(w, dst, zeros)


def _mp_body(y, src, dst, w, out, srca, wa, rows0, rows1, idxs0, idxs1, acc,
             semg, semi, semw):
    cc = lax.axis_index("c")
    sid = lax.axis_index("s")
    wid = cc * NS + sid
    rows = (rows0, rows1)
    idxs = (idxs0, idxs1)

    def zr(i, carry):
        for j in range(8):
            rows0[i, pl.ds(j * 16, 16)] = jnp.zeros((16,), _f32)
        return carry
    lax.fori_loop(0, CH, zr, 0)

    base = sid * RPT
    for k in range(RPT // CH):
        pltpu.sync_copy(rows0, acc.at[pl.ds(base + k * CH, CH)])
    plsc.subcore_barrier()

    eoff = wid * EPT
    pltpu.sync_copy(src.at[pl.ds(eoff, EPT)], srca)
    pltpu.sync_copy(w.at[pl.ds(eoff, EPT)], wa)

    def issue_next(c, b):
        pltpu.async_copy(dst.at[pl.ds(eoff + c * CH, CH)], idxs[b],
                         semi.at[b])
        pltpu.async_copy(y.at[srca.at[pl.ds(c * CH, CH)]], rows[b],
                         semg.at[b])

    def wait_in(b):
        pltpu.make_async_copy(dst.at[pl.ds(eoff, CH)], idxs[b],
                              semi.at[b]).wait()
        pltpu.make_async_copy(y.at[srca.at[pl.ds(0, CH)]], rows[b],
                              semg.at[b]).wait()

    def scatter(b):
        pltpu.async_copy(rows[b], acc.at[idxs[b]], semw.at[b], add=True)

    def wait_scatter(b):
        pltpu.make_async_copy(rows[b], acc.at[idxs[b]], semw.at[b]).wait()

    def scale(c, b):
        rb = rows[b]

        def group(g, c2):
            wvec = wa[pl.ds(c * CH + g * 16, 16)]
            for lane in range(16):
                ws = wvec[lane]
                e = g * 16 + lane
                for j in range(8):
                    s = pl.ds(j * 16, 16)
                    rb[e, s] = rb[e, s] * ws
            return c2
        lax.fori_loop(0, CH // 16, group, 0)

    issue_next(0, 0)
    wait_in(0)
    scale(0, 0)
    scatter(0)
    issue_next(1, 1)

    def pair(t, carry):
        c1 = 2 * t + 1
        wait_in(1)
        scale(c1, 1)
        wait_scatter(0)
        scatter(1)
        issue_next(c1 + 1, 0)

        c2 = 2 * t + 2
        wait_in(0)
        scale(c2, 0)
        wait_scatter(1)
        scatter(0)

        @pl.when(t < (NCH - 3) // 2)
        def _():
            issue_next(c2 + 1, 1)
        return carry
    lax.fori_loop(0, (NCH - 1) // 2, pair, 0)
    wait_scatter(0)

    plsc.subcore_barrier()
    pltpu.sync_copy(acc.at[pl.ds(base, RPT)], out.at[cc, pl.ds(base, RPT)])


def _sc_mp(y, src, dst, w):
    return pl.kernel(
        _mp_body,
        out_type=jax.ShapeDtypeStruct((NC, NPAD, D), _f32),
        mesh=_mesh(),
        scratch_types=[
            pltpu.VMEM((EPT,), _i32),
            pltpu.VMEM((EPT,), _f32),
            pltpu.VMEM((CH, D), _f32),
            pltpu.VMEM((CH, D), _f32),
            pltpu.VMEM((CH,), _i32),
            pltpu.VMEM((CH,), _i32),
            pltpu.VMEM_SHARED((NPAD, D), _f32),
            pltpu.SemaphoreType.DMA((2,)),
            pltpu.SemaphoreType.DMA((2,)),
            pltpu.SemaphoreType.DMA((2,)),
        ],
    )(y, src, dst, w)


_DCH = PPT // CH


def _dec_body(p, q, i0, i1, out, i0a, i1a, ra0, ra1, ra2, ra3,
              semg, sema, semw):
    cc = lax.axis_index("c")
    sid = lax.axis_index("s")
    wid = cc * NS + sid
    poff = wid * PPT
    ra = (ra0, ra1, ra2, ra3)

    pltpu.sync_copy(i0.at[pl.ds(poff, PPT)], i0a)
    pltpu.sync_copy(i1.at[pl.ds(poff, PPT)], i1a)

    def g1(c, b):
        pltpu.async_copy(p.at[i0a.at[pl.ds(c * CH, CH)]], ra[b], semg.at[b])

    def wait_g1(b):
        pltpu.make_async_copy(p.at[i0a.at[pl.ds(0, CH)]], ra[b],
                              semg.at[b]).wait()

    def gadd(c, b):
        pltpu.async_copy(q.at[i1a.at[pl.ds(c * CH, CH)]], ra[b], sema.at[b],
                         add=True)

    def wait_gadd(b):
        pltpu.make_async_copy(q.at[i1a.at[pl.ds(0, CH)]], ra[b],
                              sema.at[b]).wait()

    def write(c, b):
        pltpu.async_copy(ra[b], out.at[pl.ds(poff + c * CH, CH)], semw.at[b])

    def wait_write(b):
        pltpu.make_async_copy(ra[b], out.at[pl.ds(poff, CH)],
                              semw.at[b]).wait()

    g1(0, 0)
    g1(1, 1)
    wait_g1(0)
    gadd(0, 0)
    g1(2, 2)

    def quad(t, carry):
        for k in range(4):
            c = 4 * t + 1 + k
            bc = (1 + k) % 4
            wait_g1(bc)
            gadd(c, bc)
            wait_gadd((bc - 1) % 4)
            write(c - 1, (bc - 1) % 4)

            @pl.when(c >= 2)
            def _():
                wait_write((bc - 2) % 4)

            @pl.when(c <= _DCH - 3)
            def _():
                g1(c + 2, (bc + 2) % 4)
        return carry
    lax.fori_loop(0, (_DCH - 1) // 4, quad, 0)

    wait_gadd((_DCH - 1) % 4)
    write(_DCH - 1, (_DCH - 1) % 4)
    wait_write((_DCH - 2) % 4)
    wait_write((_DCH - 1) % 4)


def _sc_dec(p, q, i0, i1):
    return pl.kernel(
        _dec_body,
        out_type=jax.ShapeDtypeStruct((NP2, D), _f32),
        mesh=_mesh(),
        scratch_types=[
            pltpu.VMEM((PPT,), _i32),
            pltpu.VMEM((PPT,), _i32),
            pltpu.VMEM((CH, D), _f32),
            pltpu.VMEM((CH, D), _f32),
            pltpu.VMEM((CH, D), _f32),
            pltpu.VMEM((CH, D), _f32),
            pltpu.SemaphoreType.DMA((4,)),
            pltpu.SemaphoreType.DMA((4,)),
            pltpu.SemaphoreType.DMA((4,)),
        ],
    )(p, q, i0, i1)


def _tc_xw_body(x, w1, xw_o):
    xw_o[...] = jnp.dot(x[...], w1[...], preferred_element_type=_f32)


def _tc_xw(x, w1):
    return pl.pallas_call(
        _tc_xw_body,
        out_shape=jax.ShapeDtypeStruct((NPAD, D), _f32),
    )(x, w1)


def _tc_first_body(degp, xw, dis_o, y_o):
    dis = lax.rsqrt(1.0 + degp[0] + degp[1])
    dis_o[...] = dis
    y_o[...] = xw[...] * dis


def _tc_first(degp, xw):
    return pl.pallas_call(
        _tc_first_body,
        out_shape=(jax.ShapeDtypeStruct((NPAD, 1), _f32),
                   jax.ShapeDtypeStruct((NPAD, D), _f32)),
    )(degp, xw)


def _tc_mid_body(m, y, dis, b, g, beta, w, y_o):
    h = (m[0] + m[1] + y[...]) * dis[...] + b[...]
    hv = h[:N]
    mu = jnp.mean(hv, axis=0, keepdims=True)
    var = jnp.mean((hv - mu) ** 2, axis=0, keepdims=True)
    hn = (h - mu) * lax.rsqrt(var + EPS) * g[...] + beta[...]
    hn = jnp.maximum(hn, 0.0)
    y_o[...] = jnp.dot(hn, w[...], preferred_element_type=_f32) * dis[...]


def _tc_mid(m, y, dis, b, g, beta, w):
    return pl.pallas_call(
        _tc_mid_body,
        out_shape=jax.ShapeDtypeStruct((NPAD, D), _f32),
    )(m, y, dis, b, g, beta, w)


def _tc_enc_body(m, y, dis, b, dw1a, dw1b, db1, p_o, q_o):
    enc = (m[0] + m[1] + y[...]) * dis[...] + b[...]
    p_o[...] = jnp.dot(enc, dw1a[...], preferred_element_type=_f32) + db1[...]
    q_o[...] = jnp.dot(enc, dw1b[...], preferred_element_type=_f32)


def _tc_enc(m, y, dis, b, dw1a, dw1b, db1):
    return pl.pallas_call(
        _tc_enc_body,
        out_shape=(jax.ShapeDtypeStruct((NPAD, D), _f32),
                   jax.ShapeDtypeStruct((NPAD, D), _f32)),
    )(m, y, dis, b, dw1a, dw1b, db1)


_DEC_R = 1280


def _tc_dec_body(h1, dw2, db2, dw3, db3, o):
    h = jnp.maximum(h1[...], 0.0).astype(jnp.bfloat16)
    t = jnp.dot(h, dw2[...], preferred_element_type=_f32) + db2[...]
    t = jnp.maximum(t, 0.0).astype(jnp.bfloat16)
    o[...] = jnp.dot(t, dw3[...], preferred_element_type=_f32) + db3[...]


def _tc_dec(h1, dw2, db2, dw3, db3):
    grid = NP2 // _DEC_R
    return pl.pallas_call(
        _tc_dec_body,
        grid=(grid,),
        in_specs=[
            pl.BlockSpec((_DEC_R, D), lambda i: (i, 0)),
            pl.BlockSpec((D, D), lambda i: (0, 0)),
            pl.BlockSpec((1, D), lambda i: (0, 0)),
            pl.BlockSpec((D, 1), lambda i: (0, 0)),
            pl.BlockSpec((1, 1), lambda i: (0, 0)),
        ],
        out_specs=pl.BlockSpec((_DEC_R, 1), lambda i: (i, 0)),
        out_shape=jax.ShapeDtypeStruct((NP2, 1), _f32),
    )(h1, dw2, db2, dw3, db3)


def kernel(x, edge_weight, W1, b1, W2, b2, W3, b3, g1, beta1, g2, beta2,
           dW1, db1, dW2, db2, dW3, db3, edge_index,
           edge_label_index_only, neg_edge_index):
    src = edge_index[0].astype(_i32)
    dst = edge_index[1].astype(_i32)
    i0 = jnp.concatenate(
        [edge_label_index_only[0], neg_edge_index[0]]).astype(_i32)
    i1 = jnp.concatenate(
        [edge_label_index_only[1], neg_edge_index[1]]).astype(_i32)

    xp = jnp.pad(x, ((0, NPAD - N), (0, 0)))

    xw1 = _tc_xw(xp, W1)
    degp = _sc_deg(edge_weight, dst, jnp.zeros((N,), _f32))
    degp = jnp.pad(degp, ((0, 0), (0, NPAD - N))).reshape(NC, NPAD, 1)
    dis, y1 = _tc_first(degp, xw1)

    m1 = _sc_mp(y1, src, dst, edge_weight)
    y2 = _tc_mid(m1, y1, dis, b1.reshape(1, D), g1.reshape(1, D),
                 beta1.reshape(1, D), W2)
    m2 = _sc_mp(y2, src, dst, edge_weight)
    y3 = _tc_mid(m2, y2, dis, b2.reshape(1, D), g2.reshape(1, D),
                 beta2.reshape(1, D), W3)
    m3 = _sc_mp(y3, src, dst, edge_weight)

    p, q = _tc_enc(m3, y3, dis, b3.reshape(1, D), dW1[:D], dW1[D:],
                   db1.reshape(1, D))
    h1 = _sc_dec(p, q, i0, i1)
    out = _tc_dec(h1, dW2.astype(jnp.bfloat16), db2.reshape(1, D),
                  dW3.astype(jnp.bfloat16), db3.reshape(1, 1))
    return out[:, 0]
